# baseline (speedup 1.0000x reference)
import functools

import jax
import jax.numpy as jnp
from jax import lax
from jax.experimental import pallas as pl
from jax.experimental.pallas import tpu as pltpu
from jax.experimental.pallas import tpu_sc as plsc

VOCAB_SIZE = 1000000
EMBED_DIM = 64
BATCH = 16384

NUM_CORES = 2
NUM_SUBCORES = 16
NUM_WORKERS = NUM_CORES * NUM_SUBCORES
B_PER_W = BATCH // NUM_WORKERS
CHUNK = 128
N_CHUNKS = B_PER_W // CHUNK


def _gather_body(tgt_hbm, oth_hbm, table_hbm, out_t_hbm, out_o_hbm,
                 idx_t, idx_o, rows_t, rows_o, sem_t, sem_o):
  wid = lax.axis_index("s") * NUM_CORES + lax.axis_index("c")
  base = wid * B_PER_W
  pltpu.sync_copy(tgt_hbm.at[pl.ds(base, B_PER_W)], idx_t)
  pltpu.sync_copy(oth_hbm.at[pl.ds(base, B_PER_W)], idx_o)
  waits = []
  for j in range(N_CHUNKS):
    sl = pl.ds(j * CHUNK, CHUNK)
    waits.append(pltpu.async_copy(
        table_hbm.at[idx_t.at[sl]], rows_t.at[sl], sem_t))
  for j in range(N_CHUNKS):
    sl = pl.ds(j * CHUNK, CHUNK)
    waits.append(pltpu.async_copy(
        table_hbm.at[idx_o.at[sl]], rows_o.at[sl], sem_o))
  for w in waits:
    w.wait()
  pltpu.sync_copy(rows_t, out_t_hbm.at[pl.ds(base, B_PER_W)])
  pltpu.sync_copy(rows_o, out_o_hbm.at[pl.ds(base, B_PER_W)])


@jax.jit
def kernel(target, other, embed_table):
  mesh = plsc.VectorSubcoreMesh(
      core_axis_name="c", subcore_axis_name="s",
      num_cores=NUM_CORES, num_subcores=NUM_SUBCORES)
  run = pl.kernel(
      _gather_body,
      out_type=(
          jax.ShapeDtypeStruct((BATCH, EMBED_DIM), jnp.float32),
          jax.ShapeDtypeStruct((BATCH, EMBED_DIM), jnp.float32),
      ),
      mesh=mesh,
      scratch_types=[
          pltpu.VMEM((B_PER_W,), jnp.int32),
          pltpu.VMEM((B_PER_W,), jnp.int32),
          pltpu.VMEM((B_PER_W, EMBED_DIM), jnp.float32),
          pltpu.VMEM((B_PER_W, EMBED_DIM), jnp.float32),
          pltpu.SemaphoreType.DMA,
          pltpu.SemaphoreType.DMA,
      ],
      compiler_params=pltpu.CompilerParams(use_tc_tiling_on_sc=False),
  )
  return run(target.astype(jnp.int32), other.astype(jnp.int32), embed_table)
